# baseline (speedup 1.0000x reference)
import functools

import jax
import jax.numpy as jnp
from jax import lax
from jax.experimental import pallas as pl
from jax.experimental.pallas import tpu as pltpu
from jax.experimental.pallas import tpu_sc as plsc


def _sc_geometry():
    try:
        info = plsc.get_sparse_core_info()
        return info.num_cores, info.num_subcores
    except Exception:
        return 2, 16

_LANES = 16
_CHUNK = 128


def _gather_body(n_rows, d, b_per_w, nc, idx_hbm, flat_hbm, out_hbm,
                 idx_v, idx_col, outt_v, sem):
    wid = lax.axis_index("s") * nc + lax.axis_index("c")
    base = wid * b_per_w
    n_chunks = b_per_w // _CHUNK
    pltpu.sync_copy(idx_hbm.at[pl.ds(base, b_per_w)], idx_v)

    def col_body(c):
        shift = c * n_rows
        for q in range(b_per_w // _LANES):
            vec = idx_v[pl.ds(q * _LANES, _LANES)] + shift
            j, r = divmod(q, _CHUNK // _LANES)
            idx_col[j, pl.ds(r * _LANES, _LANES)] = vec
        copies = [
            pltpu.async_copy(
                flat_hbm.at[idx_col.at[j]],
                outt_v.at[c, pl.ds(j * _CHUNK, _CHUNK)],
                sem,
            )
            for j in range(n_chunks)
        ]
        for cp in copies:
            cp.wait()

    pl.loop(0, d)(col_body)
    pltpu.sync_copy(outt_v, out_hbm.at[:, pl.ds(base, b_per_w)])


def kernel(idx, z):
    (batch,) = idx.shape
    n, d = z.shape
    nc, ns = _sc_geometry()
    nw = nc * ns
    b_per_w = batch // nw
    idx1 = jnp.asarray(idx, jnp.int32)
    flat = z.T.reshape(n * d)

    mesh = plsc.VectorSubcoreMesh(core_axis_name="c", subcore_axis_name="s")
    run = functools.partial(
        pl.kernel,
        out_type=jax.ShapeDtypeStruct((d, batch), jnp.float32),
        mesh=mesh,
        scratch_types=[
            pltpu.VMEM((b_per_w,), jnp.int32),
            pltpu.VMEM((b_per_w // _CHUNK, _CHUNK), jnp.int32),
            pltpu.VMEM((d, b_per_w), jnp.float32),
            pltpu.SemaphoreType.DMA,
        ],
    )(functools.partial(_gather_body, n, d, b_per_w, nc))
    outt = run(idx1, flat)
    return outt.T


# speedup vs baseline: 1.0109x; 1.0109x over previous
import functools

import jax
import jax.numpy as jnp
from jax import lax
from jax.experimental import pallas as pl
from jax.experimental.pallas import tpu as pltpu
from jax.experimental.pallas import tpu_sc as plsc


def _sc_geometry():
    try:
        info = plsc.get_sparse_core_info()
        return info.num_cores, info.num_subcores
    except Exception:
        return 2, 16

_LANES = 16
_CHUNK = 128
_LAG = 4


def _gather_body(n_rows, d, b_per_w, nc, idx_hbm, flat_hbm, out_hbm,
                 idx_v, idx_all, outt_v, sems):
    wid = lax.axis_index("s") * nc + lax.axis_index("c")
    base = wid * b_per_w
    n_chunks = b_per_w // _CHUNK
    pltpu.sync_copy(idx_hbm.at[pl.ds(base, b_per_w)], idx_v)

    def build_body(c):
        shift = c * n_rows
        for q in range(b_per_w // _LANES):
            vec = idx_v[pl.ds(q * _LANES, _LANES)] + shift
            j, r = divmod(q, _CHUNK // _LANES)
            idx_all[c, j, pl.ds(r * _LANES, _LANES)] = vec

    pl.loop(0, d)(build_body)

    col_bytes = 4 * b_per_w

    def fire_body(c):
        slot = lax.rem(c, _LAG)

        @pl.when(c < d)
        def _fire():
            for j in range(n_chunks):
                pltpu.async_copy(
                    flat_hbm.at[idx_all.at[c, j]],
                    outt_v.at[c, pl.ds(j * _CHUNK, _CHUNK)],
                    sems.at[slot],
                )

        @pl.when(c >= _LAG)
        def _drain():
            cprev = lax.max(c - _LAG, 0)
            pltpu.make_async_copy(
                flat_hbm.at[pl.ds(0, b_per_w)],
                outt_v.at[cprev],
                sems.at[slot],
            ).wait()

    pl.loop(0, d + _LAG)(fire_body)
    pltpu.sync_copy(outt_v, out_hbm.at[:, pl.ds(base, b_per_w)])


def kernel(idx, z):
    (batch,) = idx.shape
    n, d = z.shape
    nc, ns = _sc_geometry()
    nw = nc * ns
    b_per_w = batch // nw
    idx1 = jnp.asarray(idx, jnp.int32)
    flat = z.T.reshape(n * d)

    mesh = plsc.VectorSubcoreMesh(core_axis_name="c", subcore_axis_name="s")
    run = functools.partial(
        pl.kernel,
        out_type=jax.ShapeDtypeStruct((d, batch), jnp.float32),
        mesh=mesh,
        scratch_types=[
            pltpu.VMEM((b_per_w,), jnp.int32),
            pltpu.VMEM((d, b_per_w // _CHUNK, _CHUNK), jnp.int32),
            pltpu.VMEM((d, b_per_w), jnp.float32),
            pltpu.SemaphoreType.DMA((_LAG,)),
        ],
    )(functools.partial(_gather_body, n, d, b_per_w, nc))
    outt = run(idx1, flat)
    return outt.T


# speedup vs baseline: 4.0608x; 4.0170x over previous
import functools

import jax
import jax.numpy as jnp
from jax import lax
from jax.experimental import pallas as pl
from jax.experimental.pallas import tpu as pltpu
from jax.experimental.pallas import tpu_sc as plsc


def _sc_geometry():
    try:
        info = plsc.get_sparse_core_info()
        return info.num_cores, info.num_subcores
    except Exception:
        return 2, 16

_LANES = 16
_CHUNK = 128
_TBLK = 512


def _repack_body(zt_ref, zpk_ref):
    x = zt_ref[...]
    y = jnp.transpose(x, (1, 0))
    h = _TBLK // 2
    zpk_ref[...] = jnp.concatenate([y[:h, :], y[h:, :]], axis=1)


def _repack(zt, n):
    grid = (n + _TBLK - 1) // _TBLK
    d = zt.shape[0]
    return pl.pallas_call(
        _repack_body,
        grid=(grid,),
        in_specs=[pl.BlockSpec((d, _TBLK), lambda j: (0, j))],
        out_specs=pl.BlockSpec((_TBLK // 2, 128), lambda j: (j, 0)),
        out_shape=jax.ShapeDtypeStruct((grid * (_TBLK // 2), 128), jnp.float32),
        compiler_params=pltpu.CompilerParams(
            dimension_semantics=("arbitrary",),
        ),
    )(zt)


def _gather_body(d, b_per_w, nc, idx_hbm, zpk_hbm, out_hbm,
                 idx_v, idxp, packed_v, outt_v, sem):
    wid = lax.axis_index("s") * nc + lax.axis_index("c")
    base = wid * b_per_w
    n_chunks = b_per_w // _CHUNK
    pltpu.sync_copy(idx_hbm.at[pl.ds(base, b_per_w)], idx_v)

    for q in range(b_per_w // _LANES):
        vec = idx_v[pl.ds(q * _LANES, _LANES)]
        row = lax.shift_left(lax.shift_right_logical(vec, 9), 8) + \
            lax.bitwise_and(vec, 255)
        j, r = divmod(q, _CHUNK // _LANES)
        idxp[j, pl.ds(r * _LANES, _LANES)] = row

    copies = [
        pltpu.async_copy(
            zpk_hbm.at[idxp.at[j]],
            packed_v.at[pl.ds(j * _CHUNK, _CHUNK)],
            sem,
        )
        for j in range(n_chunks)
    ]
    for cp in copies:
        cp.wait()

    lanes = lax.iota(jnp.int32, _LANES)

    def col_body(c):
        cvec = jnp.full((_LANES,), 0, jnp.int32) + c
        for q in range(b_per_w // _LANES):
            rows = q * _LANES + lanes
            half = lax.bitwise_and(
                lax.shift_right_logical(idx_v[pl.ds(q * _LANES, _LANES)], 8), 1)
            vals = plsc.load_gather(packed_v, [rows, half * d + cvec])
            plsc.store_scatter(outt_v, [cvec, rows], vals)

    pl.loop(0, d)(col_body)
    pltpu.sync_copy(outt_v, out_hbm.at[:, pl.ds(base, b_per_w)])


def kernel(idx, z):
    (batch,) = idx.shape
    n, d = z.shape
    nc, ns = _sc_geometry()
    nw = nc * ns
    b_per_w = batch // nw
    idx1 = jnp.asarray(idx, jnp.int32)
    zpk = _repack(z.T, n)

    mesh = plsc.VectorSubcoreMesh(core_axis_name="c", subcore_axis_name="s")
    run = functools.partial(
        pl.kernel,
        out_type=jax.ShapeDtypeStruct((d, batch), jnp.float32),
        mesh=mesh,
        scratch_types=[
            pltpu.VMEM((b_per_w,), jnp.int32),
            pltpu.VMEM((b_per_w // _CHUNK, _CHUNK), jnp.int32),
            pltpu.VMEM((b_per_w, 2 * d), jnp.float32),
            pltpu.VMEM((d, b_per_w), jnp.float32),
            pltpu.SemaphoreType.DMA,
        ],
        compiler_params=pltpu.CompilerParams(needs_layout_passes=False),
    )(functools.partial(_gather_body, d, b_per_w, nc))
    outt = run(idx1, zpk)
    return outt.T


# speedup vs baseline: 13.1977x; 3.2500x over previous
import functools

import jax
import jax.numpy as jnp
from jax import lax
from jax.experimental import pallas as pl
from jax.experimental.pallas import tpu as pltpu
from jax.experimental.pallas import tpu_sc as plsc


def _sc_geometry():
    try:
        info = plsc.get_sparse_core_info()
        return info.num_cores, info.num_subcores
    except Exception:
        return 2, 16

_LANES = 16
_CHUNK = 128
_TBLK = 4096
_TSH = _TBLK.bit_length() - 1
_HMSK = _TBLK // 2 - 1


def _repack_body(zt_ref, zpk_ref):
    x = zt_ref[...]
    y = jnp.transpose(x, (1, 0))
    h = _TBLK // 2
    zpk_ref[...] = jnp.concatenate([y[:h, :], y[h:, :]], axis=1)


def _repack(zt, n):
    grid = (n + _TBLK - 1) // _TBLK
    d = zt.shape[0]
    return pl.pallas_call(
        _repack_body,
        grid=(grid,),
        in_specs=[pl.BlockSpec((d, _TBLK), lambda j: (0, j))],
        out_specs=pl.BlockSpec((_TBLK // 2, 128), lambda j: (j, 0)),
        out_shape=jax.ShapeDtypeStruct((grid * (_TBLK // 2), 128), jnp.float32),
        compiler_params=pltpu.CompilerParams(
            dimension_semantics=("arbitrary",),
        ),
    )(zt)


def _gather_body(d, b_per_w, nc, idx_hbm, zpk_hbm, out_hbm,
                 idx_v, idxp, packed_v, outt_v, sem):
    wid = lax.axis_index("s") * nc + lax.axis_index("c")
    base = wid * b_per_w
    n_chunks = b_per_w // _CHUNK
    pltpu.sync_copy(idx_hbm.at[pl.ds(base, b_per_w)], idx_v)

    for q in range(b_per_w // _LANES):
        vec = idx_v[pl.ds(q * _LANES, _LANES)]
        row = lax.shift_left(lax.shift_right_logical(vec, _TSH), _TSH - 1) + \
            lax.bitwise_and(vec, _HMSK)
        j, r = divmod(q, _CHUNK // _LANES)
        idxp[j, pl.ds(r * _LANES, _LANES)] = row

    copies = [
        pltpu.async_copy(
            zpk_hbm.at[idxp.at[j]],
            packed_v.at[pl.ds(j * _CHUNK, _CHUNK)],
            sem,
        )
        for j in range(n_chunks)
    ]
    for cp in copies:
        cp.wait()

    lanes = lax.iota(jnp.int32, _LANES)

    def col_body(c):
        cvec = jnp.full((_LANES,), 0, jnp.int32) + c
        for q in range(b_per_w // _LANES):
            rows = q * _LANES + lanes
            half = lax.bitwise_and(
                lax.shift_right_logical(
                    idx_v[pl.ds(q * _LANES, _LANES)], _TSH - 1), 1)
            vals = plsc.load_gather(packed_v, [rows, half * d + cvec])
            plsc.store_scatter(outt_v, [cvec, rows], vals)

    pl.loop(0, d)(col_body)
    pltpu.sync_copy(outt_v, out_hbm.at[:, pl.ds(base, b_per_w)])


def kernel(idx, z):
    (batch,) = idx.shape
    n, d = z.shape
    nc, ns = _sc_geometry()
    nw = nc * ns
    b_per_w = batch // nw
    idx1 = jnp.asarray(idx, jnp.int32)
    zpk = _repack(z.T, n)

    mesh = plsc.VectorSubcoreMesh(core_axis_name="c", subcore_axis_name="s")
    run = functools.partial(
        pl.kernel,
        out_type=jax.ShapeDtypeStruct((d, batch), jnp.float32),
        mesh=mesh,
        scratch_types=[
            pltpu.VMEM((b_per_w,), jnp.int32),
            pltpu.VMEM((b_per_w // _CHUNK, _CHUNK), jnp.int32),
            pltpu.VMEM((b_per_w, 2 * d), jnp.float32),
            pltpu.VMEM((d, b_per_w), jnp.float32),
            pltpu.SemaphoreType.DMA,
        ],
        compiler_params=pltpu.CompilerParams(needs_layout_passes=False),
    )(functools.partial(_gather_body, d, b_per_w, nc))
    outt = run(idx1, zpk)
    return outt.T


# speedup vs baseline: 17.7612x; 1.3458x over previous
import functools

import jax
import jax.numpy as jnp
from jax import lax
from jax.experimental import pallas as pl
from jax.experimental.pallas import tpu as pltpu
from jax.experimental.pallas import tpu_sc as plsc


def _sc_geometry():
    try:
        info = plsc.get_sparse_core_info()
        return info.num_cores, info.num_subcores
    except Exception:
        return 2, 16

_LANES = 16
_CHUNK = 128
_TBLK = 16384
_TSH = _TBLK.bit_length() - 1
_HMSK = _TBLK // 2 - 1


def _repack_body(zt_ref, zpk_ref):
    x = zt_ref[...]
    y = jnp.transpose(x, (1, 0))
    h = _TBLK // 2
    zpk_ref[...] = jnp.concatenate([y[:h, :], y[h:, :]], axis=1)


def _repack(zt, n):
    grid = (n + _TBLK - 1) // _TBLK
    d = zt.shape[0]
    return pl.pallas_call(
        _repack_body,
        grid=(grid,),
        in_specs=[pl.BlockSpec((d, _TBLK), lambda j: (0, j))],
        out_specs=pl.BlockSpec((_TBLK // 2, 128), lambda j: (j, 0)),
        out_shape=jax.ShapeDtypeStruct((grid * (_TBLK // 2), 128), jnp.float32),
        compiler_params=pltpu.CompilerParams(
            dimension_semantics=("arbitrary",),
        ),
    )(zt)


def _gather_body(d, b_per_w, nc, idx_hbm, zpk_hbm, out_hbm,
                 idx_v, idxp, packed_v, outt_v, sem):
    wid = lax.axis_index("s") * nc + lax.axis_index("c")
    base = wid * b_per_w
    n_chunks = b_per_w // _CHUNK
    pltpu.sync_copy(idx_hbm.at[pl.ds(base, b_per_w)], idx_v)

    for q in range(b_per_w // _LANES):
        vec = idx_v[pl.ds(q * _LANES, _LANES)]
        row = lax.shift_left(lax.shift_right_logical(vec, _TSH), _TSH - 1) + \
            lax.bitwise_and(vec, _HMSK)
        j, r = divmod(q, _CHUNK // _LANES)
        idxp[j, pl.ds(r * _LANES, _LANES)] = row

    copies = [
        pltpu.async_copy(
            zpk_hbm.at[idxp.at[j]],
            packed_v.at[pl.ds(j * _CHUNK, _CHUNK)],
            sem,
        )
        for j in range(n_chunks)
    ]
    for cp in copies:
        cp.wait()

    lanes = lax.iota(jnp.int32, _LANES)

    def col_body(c):
        cvec = jnp.full((_LANES,), 0, jnp.int32) + c
        for q in range(b_per_w // _LANES):
            rows = q * _LANES + lanes
            half = lax.bitwise_and(
                lax.shift_right_logical(
                    idx_v[pl.ds(q * _LANES, _LANES)], _TSH - 1), 1)
            vals = plsc.load_gather(packed_v, [rows, half * d + cvec])
            plsc.store_scatter(outt_v, [cvec, rows], vals)

    pl.loop(0, d)(col_body)
    pltpu.sync_copy(outt_v, out_hbm.at[:, pl.ds(base, b_per_w)])


def kernel(idx, z):
    (batch,) = idx.shape
    n, d = z.shape
    nc, ns = _sc_geometry()
    nw = nc * ns
    b_per_w = batch // nw
    idx1 = jnp.asarray(idx, jnp.int32)
    zpk = _repack(z.T, n)

    mesh = plsc.VectorSubcoreMesh(core_axis_name="c", subcore_axis_name="s")
    run = functools.partial(
        pl.kernel,
        out_type=jax.ShapeDtypeStruct((d, batch), jnp.float32),
        mesh=mesh,
        scratch_types=[
            pltpu.VMEM((b_per_w,), jnp.int32),
            pltpu.VMEM((b_per_w // _CHUNK, _CHUNK), jnp.int32),
            pltpu.VMEM((b_per_w, 2 * d), jnp.float32),
            pltpu.VMEM((d, b_per_w), jnp.float32),
            pltpu.SemaphoreType.DMA,
        ],
        compiler_params=pltpu.CompilerParams(needs_layout_passes=False),
    )(functools.partial(_gather_body, d, b_per_w, nc))
    outt = run(idx1, zpk)
    return outt.T


# speedup vs baseline: 18.6678x; 1.0510x over previous
import functools

import jax
import jax.numpy as jnp
from jax import lax
from jax.experimental import pallas as pl
from jax.experimental.pallas import tpu as pltpu
from jax.experimental.pallas import tpu_sc as plsc


def _sc_geometry():
    try:
        info = plsc.get_sparse_core_info()
        return info.num_cores, info.num_subcores
    except Exception:
        return 2, 16

_LANES = 16
_CHUNK = 128
_TBLK = 32768
_TSH = _TBLK.bit_length() - 1
_HMSK = _TBLK // 2 - 1


def _repack_body(zt_ref, zpk_ref):
    x = zt_ref[...]
    y = jnp.transpose(x, (1, 0))
    h = _TBLK // 2
    zpk_ref[...] = jnp.concatenate([y[:h, :], y[h:, :]], axis=1)


def _repack(zt, n):
    grid = (n + _TBLK - 1) // _TBLK
    d = zt.shape[0]
    return pl.pallas_call(
        _repack_body,
        grid=(grid,),
        in_specs=[pl.BlockSpec((d, _TBLK), lambda j: (0, j))],
        out_specs=pl.BlockSpec((_TBLK // 2, 128), lambda j: (j, 0)),
        out_shape=jax.ShapeDtypeStruct((grid * (_TBLK // 2), 128), jnp.float32),
        compiler_params=pltpu.CompilerParams(
            dimension_semantics=("arbitrary",),
        ),
    )(zt)


def _gather_body(d, b_per_w, nc, idx_hbm, zpk_hbm, out_hbm,
                 idx_v, idxp, packed_v, outt_v, sem):
    wid = lax.axis_index("s") * nc + lax.axis_index("c")
    base = wid * b_per_w
    n_chunks = b_per_w // _CHUNK
    pltpu.sync_copy(idx_hbm.at[pl.ds(base, b_per_w)], idx_v)

    for q in range(b_per_w // _LANES):
        vec = idx_v[pl.ds(q * _LANES, _LANES)]
        row = lax.shift_left(lax.shift_right_logical(vec, _TSH), _TSH - 1) + \
            lax.bitwise_and(vec, _HMSK)
        j, r = divmod(q, _CHUNK // _LANES)
        idxp[j, pl.ds(r * _LANES, _LANES)] = row

    copies = [
        pltpu.async_copy(
            zpk_hbm.at[idxp.at[j]],
            packed_v.at[pl.ds(j * _CHUNK, _CHUNK)],
            sem,
        )
        for j in range(n_chunks)
    ]
    for cp in copies:
        cp.wait()

    lanes = lax.iota(jnp.int32, _LANES)

    def col_body(c):
        cvec = jnp.full((_LANES,), 0, jnp.int32) + c
        for q in range(b_per_w // _LANES):
            rows = q * _LANES + lanes
            half = lax.bitwise_and(
                lax.shift_right_logical(
                    idx_v[pl.ds(q * _LANES, _LANES)], _TSH - 1), 1)
            vals = plsc.load_gather(packed_v, [rows, half * d + cvec])
            plsc.store_scatter(outt_v, [cvec, rows], vals)

    pl.loop(0, d)(col_body)
    pltpu.sync_copy(outt_v, out_hbm.at[:, pl.ds(base, b_per_w)])


def kernel(idx, z):
    (batch,) = idx.shape
    n, d = z.shape
    nc, ns = _sc_geometry()
    nw = nc * ns
    b_per_w = batch // nw
    idx1 = jnp.asarray(idx, jnp.int32)
    zpk = _repack(z.T, n)

    mesh = plsc.VectorSubcoreMesh(core_axis_name="c", subcore_axis_name="s")
    run = functools.partial(
        pl.kernel,
        out_type=jax.ShapeDtypeStruct((d, batch), jnp.float32),
        mesh=mesh,
        scratch_types=[
            pltpu.VMEM((b_per_w,), jnp.int32),
            pltpu.VMEM((b_per_w // _CHUNK, _CHUNK), jnp.int32),
            pltpu.VMEM((b_per_w, 2 * d), jnp.float32),
            pltpu.VMEM((d, b_per_w), jnp.float32),
            pltpu.SemaphoreType.DMA,
        ],
        compiler_params=pltpu.CompilerParams(needs_layout_passes=False),
    )(functools.partial(_gather_body, d, b_per_w, nc))
    outt = run(idx1, zpk)
    return outt.T
